# baseline (speedup 1.0000x reference)
import jax
import jax.numpy as jnp
from jax.experimental import pallas as pl
from jax.experimental.pallas import tpu as pltpu

B, N, L, D = 4, 4096, 12, 64
LD = L * D
BD = B * D
CH = 128
NCH = N // CH
GRID = 2 * NCH + 1

_bf16 = jnp.bfloat16
_f32 = jnp.float32


def _mega_kernel(adj_any, x_ref, wm2_ref, bm2_ref, wg1_ref, bt1_ref,
                 wg2_ref, bt2_ref, wg3_ref, bt3_ref, wm1_ref, bm1_ref,
                 o_ref, adj_bf, h0, z, acc1, stage, sems):
    i = pl.program_id(0)

    def start_adj_copy(ci, slot):
        pltpu.make_async_copy(
            adj_any.at[:, pl.ds(ci * CH, CH)], stage.at[slot],
            sems.at[slot]).start()

    def wait_adj_copy(ci, slot):
        pltpu.make_async_copy(
            adj_any.at[:, pl.ds(ci * CH, CH)], stage.at[slot],
            sems.at[slot]).wait()

    @pl.when(i == 0)
    def _prologue():
        start_adj_copy(0, 0)

    def phase1(slot):
        ci = i

        @pl.when(ci + 1 < NCH)
        def _():
            start_adj_copy(ci + 1, 1 - slot)

        wait_adj_copy(ci, slot)
        rows = pl.ds(ci * CH, CH)
        cols = pl.ds(ci * CH, CH)

        a_cols = stage[slot].astype(_bf16)
        adj_bf[:, cols] = a_cols

        wm2 = wm2_ref[:].astype(_bf16)
        wg1 = wg1_ref[:].astype(_bf16)
        bm2 = bm2_ref[:]
        for bi in range(B):
            xb = x_ref[bi].astype(_bf16)
            hc = jnp.maximum(
                jnp.dot(xb, wm2, preferred_element_type=_f32) + bm2, 0.0)
            hcb = hc.astype(_bf16)
            bsl = slice(bi * D, (bi + 1) * D)
            h0[rows, bsl] = hcb
            z[rows, bsl] = jnp.dot(hcb, wg1,
                                   preferred_element_type=_f32).astype(_bf16)

        contrib = jnp.dot(a_cols, z[rows, :], preferred_element_type=_f32)

        @pl.when(ci == 0)
        def _():
            acc1[:] = contrib

        @pl.when(ci > 0)
        def _():
            acc1[:] = acc1[:] + contrib

    @pl.when(jnp.logical_and(i < NCH, i % 2 == 0))
    def _p1_even():
        phase1(0)

    @pl.when(jnp.logical_and(i < NCH, i % 2 == 1))
    def _p1_odd():
        phase1(1)

    @pl.when(i == NCH)
    def _phase2():
        bt1 = bt1_ref[:]
        bt2 = bt2_ref[:]
        wg2 = wg2_ref[:].astype(_bf16)
        wg3 = wg3_ref[:].astype(_bf16)
        for rc in range(4):
            rs = slice(rc * (N // 4), (rc + 1) * (N // 4))
            h0[rs, :] = jnp.maximum(acc1[rs, :] + bt1, 0.0).astype(_bf16)
        for bi in range(B):
            bsl = slice(bi * D, (bi + 1) * D)
            z[:, bsl] = jnp.dot(h0[:, bsl], wg2,
                                preferred_element_type=_f32).astype(_bf16)
        for rc in range(4):
            rs = slice(rc * (N // 4), (rc + 1) * (N // 4))
            agg = jnp.dot(adj_bf[rs, :], z[:],
                          preferred_element_type=_f32)
            acc1[rs, :] = agg
        for rc in range(4):
            rs = slice(rc * (N // 4), (rc + 1) * (N // 4))
            h0[rs, :] = jnp.maximum(acc1[rs, :] + bt2, 0.0).astype(_bf16)
        for bi in range(B):
            bsl = slice(bi * D, (bi + 1) * D)
            z[:, bsl] = jnp.dot(h0[:, bsl], wg3,
                                preferred_element_type=_f32).astype(_bf16)

    @pl.when(i > NCH)
    def _phase3():
        ri = i - NCH - 1
        rows = pl.ds(ri * CH, CH)
        bt3 = bt3_ref[:]
        bm1 = bm1_ref[:]
        wm1 = wm1_ref[:].astype(_bf16)
        agg = jnp.dot(adj_bf[rows, :], z[:], preferred_element_type=_f32)
        h3c = jnp.maximum(agg + bt3, 0.0).astype(_bf16)
        for bi in range(B):
            hb = h3c[:, bi * D:(bi + 1) * D]
            o = jnp.dot(hb, wm1, preferred_element_type=_f32) + bm1
            o_ref[bi] = jnp.maximum(o, 0.0)


def kernel(x, adj, W_mlp2, b_mlp2, W_g1, b_g1, W_g2, b_g2, W_g3, b_g3,
           W_mlp1, b_mlp1):
    xf = x.reshape(B, N, LD)
    bm2 = b_mlp2.reshape(1, D)
    bt = [jnp.tile(b, B).reshape(1, BD) for b in (b_g1, b_g2, b_g3)]
    bm1 = b_mlp1.reshape(1, LD)

    last = NCH - 1

    out = pl.pallas_call(
        _mega_kernel,
        grid=(GRID,),
        in_specs=[
            pl.BlockSpec(memory_space=pltpu.MemorySpace.HBM),
            pl.BlockSpec((B, CH, LD),
                         lambda i: (0, jnp.minimum(i, last), 0)),
            pl.BlockSpec((LD, D), lambda i: (0, 0)),
            pl.BlockSpec((1, D), lambda i: (0, 0)),
            pl.BlockSpec((D, D), lambda i: (0, 0)),
            pl.BlockSpec((1, BD), lambda i: (0, 0)),
            pl.BlockSpec((D, D), lambda i: (0, 0)),
            pl.BlockSpec((1, BD), lambda i: (0, 0)),
            pl.BlockSpec((D, D), lambda i: (0, 0)),
            pl.BlockSpec((1, BD), lambda i: (0, 0)),
            pl.BlockSpec((D, LD), lambda i: (0, 0)),
            pl.BlockSpec((1, LD), lambda i: (0, 0)),
        ],
        out_specs=pl.BlockSpec(
            (B, CH, LD), lambda i: (0, jnp.maximum(i - NCH - 1, 0), 0)),
        out_shape=jax.ShapeDtypeStruct((B, N, LD), _f32),
        scratch_shapes=[
            pltpu.VMEM((N, N), _bf16),
            pltpu.VMEM((N, BD), _bf16),
            pltpu.VMEM((N, BD), _bf16),
            pltpu.VMEM((N, BD), _f32),
            pltpu.VMEM((2, N, CH), _f32),
            pltpu.SemaphoreType.DMA((2,)),
        ],
    )(adj, xf, W_mlp2, bm2, W_g1, bt[0], W_g2, bt[1], W_g3, bt[2],
      W_mlp1, bm1)
    return out
